# Initial kernel scaffold; baseline (speedup 1.0000x reference)
import jax
import jax.numpy as jnp
from jax.experimental import pallas as pl


def kernel(x, edge_index, W1l, b1, W1r, W2l, b2, W2r, Wout, bout):
    raise NotImplementedError("write your pallas kernel here")


# speedup vs baseline: 20.1746x; 20.1746x over previous
import functools

import jax
import jax.numpy as jnp
from jax import lax
from jax.experimental import pallas as pl
from jax.experimental.pallas import tpu as pltpu
from jax.experimental.pallas import tpu_sc as plsc

_NC = 2
_NS = 16
_NW = _NC * _NS


@functools.lru_cache(maxsize=None)
def _make_seg_sum(n, e, h, with_count, chunk):
    epw = e // _NW
    nchunks = epw // chunk
    assert epw * _NW == e and nchunks * chunk == epw and chunk % 8 == 0
    rps = (n + _NS - 1) // _NS
    rps = (rps + 7) & ~7
    n_pad = rps * _NS

    mesh = plsc.VectorSubcoreMesh(core_axis_name="c", subcore_axis_name="s")

    out_type = [jax.ShapeDtypeStruct((_NC, n_pad, h), jnp.float32)]
    scratch = [
        pltpu.VMEM((chunk,), jnp.int32),
        pltpu.VMEM((chunk,), jnp.int32),
        pltpu.VMEM((chunk, h), jnp.float32),
        pltpu.VMEM((rps, h), jnp.float32),
        pltpu.VMEM_SHARED((n_pad, h), jnp.float32),
        pltpu.SemaphoreType.DMA,
    ]
    if with_count:
        out_type.append(jax.ShapeDtypeStruct((_NC, n_pad, h), jnp.float32))
        scratch.append(pltpu.VMEM((chunk, h), jnp.float32))
        scratch.append(pltpu.VMEM_SHARED((n_pad, h), jnp.float32))

    def body(y_hbm, src_hbm, dst_hbm, *rest):
        if with_count:
            (out_hbm, cnt_hbm, src_v, dst_v, rows_v, zer_v, acc_sh, sem,
             ones_v, cacc_sh) = rest
        else:
            out_hbm, src_v, dst_v, rows_v, zer_v, acc_sh, sem = rest
        c = lax.axis_index("c")
        s = lax.axis_index("s")
        wid = s * _NC + c

        zrow = jnp.zeros((16,), jnp.float32)

        def zfill(i, _):
            zer_v[i, :] = zrow
            return 0

        lax.fori_loop(0, rps, zfill, 0)
        pltpu.sync_copy(zer_v, acc_sh.at[pl.ds(s * rps, rps), :])
        if with_count:
            pltpu.sync_copy(zer_v, cacc_sh.at[pl.ds(s * rps, rps), :])
            orow = jnp.ones((16,), jnp.float32)

            def ofill(i, _):
                ones_v[i, :] = orow
                return 0

            lax.fori_loop(0, chunk, ofill, 0)
        plsc.subcore_barrier()

        base = wid * epw

        def do_chunk(ci, _):
            off = base + ci * chunk
            pltpu.sync_copy(src_hbm.at[pl.ds(off, chunk)], src_v)
            pltpu.sync_copy(dst_hbm.at[pl.ds(off, chunk)], dst_v)
            pltpu.async_copy(y_hbm.at[src_v], rows_v, sem).wait()
            pltpu.sync_copy(rows_v, acc_sh.at[dst_v], add=True)
            if with_count:
                pltpu.sync_copy(ones_v, cacc_sh.at[dst_v], add=True)
            return 0

        lax.fori_loop(0, nchunks, do_chunk, 0)
        plsc.subcore_barrier()

        pltpu.sync_copy(acc_sh.at[pl.ds(s * rps, rps), :],
                        out_hbm.at[c, pl.ds(s * rps, rps), :])
        if with_count:
            pltpu.sync_copy(cacc_sh.at[pl.ds(s * rps, rps), :],
                            cnt_hbm.at[c, pl.ds(s * rps, rps), :])

    return pl.kernel(body, out_type=out_type, mesh=mesh,
                     scratch_types=scratch,
                     compiler_params=pltpu.CompilerParams(
                         use_tc_tiling_on_sc=False))


def _dot_t(a, w):
    return lax.dot_general(a, w, (((1,), (1,)), ((), ())),
                           preferred_element_type=jnp.float32)


def _stage_a(x_ref, wl_ref, wr_ref, y1_ref, xr_ref):
    x = x_ref[...]
    y1_ref[...] = _dot_t(x, wl_ref[...])
    xr_ref[...] = _dot_t(x, wr_ref[...])


def _stage_b(s1_ref, cnt_ref, xr_ref, b1_ref, w2l_ref, w2r_ref,
             y2_ref, hr2_ref):
    cnt = jnp.maximum(cnt_ref[0] + cnt_ref[1], 1.0)
    h1 = jax.nn.relu((s1_ref[0] + s1_ref[1]) / cnt + b1_ref[...]
                     + xr_ref[...])
    y2_ref[...] = _dot_t(h1, w2l_ref[...])
    hr2_ref[...] = _dot_t(h1, w2r_ref[...])


def _stage_c(s2_ref, cnt_ref, hr2_ref, b2_ref, wout_ref, bout_ref, out_ref):
    cnt = jnp.maximum(cnt_ref[0] + cnt_ref[1], 1.0)
    h2 = jax.nn.relu((s2_ref[0] + s2_ref[1]) / cnt + b2_ref[...]
                     + hr2_ref[...])
    logits = _dot_t(h2, wout_ref[...]) + bout_ref[...]
    m = jnp.max(logits, axis=-1, keepdims=True)
    lse = jnp.log(jnp.sum(jnp.exp(logits - m), axis=-1, keepdims=True)) + m
    out_ref[...] = logits - lse


def kernel(x, edge_index, W1l, b1, W1r, W2l, b2, W2r, Wout, bout):
    n, d = x.shape
    e = edge_index.shape[1]
    h = W1l.shape[0]
    c_out = Wout.shape[0]

    src = edge_index[0]
    dst = edge_index[1]
    b1r = b1.reshape(1, h)
    b2r = b2.reshape(1, h)
    boutr = bout.reshape(1, c_out)

    f32 = jnp.float32
    y1, xr = pl.pallas_call(
        _stage_a,
        out_shape=[jax.ShapeDtypeStruct((n, h), f32),
                   jax.ShapeDtypeStruct((n, h), f32)],
    )(x, W1l, W1r)

    seg1 = _make_seg_sum(n, e, h, True, 2000)
    s1, cnt = seg1(y1, src, dst)
    n_pad = s1.shape[1]

    def _full(shape):
        nd = len(shape)
        return pl.BlockSpec(shape, lambda i: (0,) * nd)

    acc_spec = pl.BlockSpec((2, n, h), lambda i: (0, 0, 0))

    y2, hr2 = pl.pallas_call(
        _stage_b,
        grid=(1,),
        out_shape=[jax.ShapeDtypeStruct((n, h), f32),
                   jax.ShapeDtypeStruct((n, h), f32)],
        in_specs=[acc_spec, acc_spec, _full((n, h)), _full((1, h)),
                  _full((h, h)), _full((h, h))],
        out_specs=[_full((n, h)), _full((n, h))],
    )(s1, cnt, xr, b1r, W2l, W2r)

    seg2 = _make_seg_sum(n, e, h, False, 2000)
    (s2,) = seg2(y2, src, dst)

    out = pl.pallas_call(
        _stage_c,
        grid=(1,),
        out_shape=jax.ShapeDtypeStruct((n, c_out), f32),
        in_specs=[acc_spec, acc_spec, _full((n, h)), _full((1, h)),
                  _full((c_out, h)), _full((1, c_out))],
        out_specs=_full((n, c_out)),
    )(s2, cnt, hr2, b2r, Wout, boutr)
    return out


# speedup vs baseline: 24.7908x; 1.2288x over previous
import functools

import jax
import jax.numpy as jnp
from jax import lax
from jax.experimental import pallas as pl
from jax.experimental.pallas import tpu as pltpu
from jax.experimental.pallas import tpu_sc as plsc

_NC = 2
_NS = 16
_NW = _NC * _NS


@functools.lru_cache(maxsize=None)
def _make_seg_sum(n, e, h, with_count, chunk):
    epw = e // _NW
    nchunks = epw // chunk
    assert epw * _NW == e and nchunks * chunk == epw and chunk % 16 == 0
    assert n % 16 == 0
    rps = (n + _NS - 1) // _NS
    rps = (rps + 15) & ~15
    n_pad = rps * _NS

    mesh = plsc.VectorSubcoreMesh(core_axis_name="c", subcore_axis_name="s")

    out_type = [jax.ShapeDtypeStruct((_NC, n_pad, h), jnp.float32)]
    scratch = [
        pltpu.VMEM((chunk,), jnp.int32),
        pltpu.VMEM((chunk,), jnp.int32),
        pltpu.VMEM((chunk,), jnp.int32),
        pltpu.VMEM((chunk,), jnp.int32),
        pltpu.VMEM((chunk, h), jnp.float32),
        pltpu.VMEM((chunk, h), jnp.float32),
        pltpu.VMEM((rps, h), jnp.float32),
        pltpu.VMEM_SHARED((n_pad, h), jnp.float32),
        pltpu.SemaphoreType.DMA,
        pltpu.SemaphoreType.DMA,
        pltpu.SemaphoreType.DMA,
        pltpu.SemaphoreType.DMA,
    ]
    if with_count:
        out_type.append(jax.ShapeDtypeStruct((_NC, n_pad), jnp.float32))
        scratch.append(pltpu.VMEM((chunk,), jnp.float32))
        scratch.append(pltpu.VMEM((rps,), jnp.float32))
        scratch.append(pltpu.VMEM_SHARED((n_pad,), jnp.float32))
        scratch.append(pltpu.SemaphoreType.DMA)
        scratch.append(pltpu.SemaphoreType.DMA)

    def body(y_hbm, src_hbm, dst_hbm, *rest):
        if with_count:
            (out_hbm, cnt_hbm, sv0, sv1, dv0, dv1, rv0, rv1, zer_v, acc_sh,
             gs0, gs1, ss0, ss1, ones_v, zc_v, cnt_sh, cs0, cs1) = rest
            csem = (cs0, cs1)
        else:
            (out_hbm, sv0, sv1, dv0, dv1, rv0, rv1, zer_v, acc_sh,
             gs0, gs1, ss0, ss1) = rest
        src_v = (sv0, sv1)
        dst_v = (dv0, dv1)
        rows_v = (rv0, rv1)
        gsem = (gs0, gs1)
        ssem = (ss0, ss1)
        c = lax.axis_index("c")
        s = lax.axis_index("s")
        wid = s * _NC + c

        zrow = jnp.zeros((16,), jnp.float32)

        def zfill(i, _):
            zer_v[i, :] = zrow
            return 0

        lax.fori_loop(0, rps, zfill, 0)
        pltpu.sync_copy(zer_v, acc_sh.at[pl.ds(s * rps, rps), :])
        orow = jnp.ones((16,), jnp.float32)
        if with_count:
            def czfill(i, _):
                zc_v[pl.ds(i * 16, 16)] = zrow
                return 0

            def ofill(i, _):
                ones_v[pl.ds(i * 16, 16)] = orow
                return 0

            lax.fori_loop(0, rps // 16, czfill, 0)
            lax.fori_loop(0, chunk // 16, ofill, 0)
            pltpu.sync_copy(zc_v, cnt_sh.at[pl.ds(s * rps, rps)])
        plsc.subcore_barrier()

        base = wid * epw

        def load_idx(i, b):
            off = base + i * chunk
            pltpu.sync_copy(src_hbm.at[pl.ds(off, chunk)], src_v[b])
            pltpu.sync_copy(dst_hbm.at[pl.ds(off, chunk)], dst_v[b])

        load_idx(0, 0)
        gathers = [pltpu.async_copy(y_hbm.at[src_v[0]], rows_v[0],
                                    gsem[0]), None]
        scatters = [None, None]
        cscatters = [None, None]
        for i in range(nchunks):
            b = i % 2
            b1 = 1 - b
            gathers[b].wait()
            scatters[b] = pltpu.async_copy(rows_v[b], acc_sh.at[dst_v[b]],
                                           ssem[b], add=True)
            if with_count:
                cscatters[b] = pltpu.async_copy(ones_v,
                                                cnt_sh.at[dst_v[b]],
                                                csem[b], add=True)
            if i + 1 < nchunks:
                if scatters[b1] is not None:
                    scatters[b1].wait()
                    scatters[b1] = None
                if cscatters[b1] is not None:
                    cscatters[b1].wait()
                    cscatters[b1] = None
                load_idx(i + 1, b1)
                gathers[b1] = pltpu.async_copy(y_hbm.at[src_v[b1]],
                                               rows_v[b1], gsem[b1])
        for sc in scatters + cscatters:
            if sc is not None:
                sc.wait()
        plsc.subcore_barrier()

        pltpu.sync_copy(acc_sh.at[pl.ds(s * rps, rps), :],
                        out_hbm.at[c, pl.ds(s * rps, rps), :])
        if with_count:
            pltpu.sync_copy(cnt_sh.at[pl.ds(s * rps, rps)],
                            cnt_hbm.at[c, pl.ds(s * rps, rps)])

    return pl.kernel(body, out_type=out_type, mesh=mesh,
                     scratch_types=scratch,
                     compiler_params=pltpu.CompilerParams(
                         use_tc_tiling_on_sc=False))


def _dot_t(a, w):
    return lax.dot_general(a, w, (((1,), (1,)), ((), ())),
                           preferred_element_type=jnp.float32)


def _stage_a(x_ref, wl_ref, wr_ref, y1_ref, xr_ref):
    x = x_ref[...]
    y1_ref[...] = _dot_t(x, wl_ref[...])
    xr_ref[...] = _dot_t(x, wr_ref[...])


def _stage_b(n, s1_ref, cnt_ref, xr_ref, b1_ref, w2l_ref, w2r_ref,
             y2_ref, hr2_ref):
    cnt = jnp.maximum(cnt_ref[0, :n] + cnt_ref[1, :n], 1.0)[:, None]
    h1 = jax.nn.relu((s1_ref[0] + s1_ref[1]) / cnt + b1_ref[...]
                     + xr_ref[...])
    y2_ref[...] = _dot_t(h1, w2l_ref[...])
    hr2_ref[...] = _dot_t(h1, w2r_ref[...])


def _stage_c(n, s2_ref, cnt_ref, hr2_ref, b2_ref, wout_ref, bout_ref,
             out_ref):
    cnt = jnp.maximum(cnt_ref[0, :n] + cnt_ref[1, :n], 1.0)[:, None]
    h2 = jax.nn.relu((s2_ref[0] + s2_ref[1]) / cnt + b2_ref[...]
                     + hr2_ref[...])
    logits = _dot_t(h2, wout_ref[...]) + bout_ref[...]
    m = jnp.max(logits, axis=-1, keepdims=True)
    lse = jnp.log(jnp.sum(jnp.exp(logits - m), axis=-1, keepdims=True)) + m
    out_ref[...] = logits - lse


def kernel(x, edge_index, W1l, b1, W1r, W2l, b2, W2r, Wout, bout):
    n, d = x.shape
    e = edge_index.shape[1]
    h = W1l.shape[0]
    c_out = Wout.shape[0]

    src = edge_index[0]
    dst = edge_index[1]
    b1r = b1.reshape(1, h)
    b2r = b2.reshape(1, h)
    boutr = bout.reshape(1, c_out)

    f32 = jnp.float32
    y1, xr = pl.pallas_call(
        _stage_a,
        out_shape=[jax.ShapeDtypeStruct((n, h), f32),
                   jax.ShapeDtypeStruct((n, h), f32)],
    )(x, W1l, W1r)

    seg1 = _make_seg_sum(n, e, h, True, 2000)
    s1, cnt = seg1(y1, src, dst)
    n_pad = s1.shape[1]

    def _full(shape):
        nd = len(shape)
        return pl.BlockSpec(shape, lambda i: (0,) * nd)

    acc_spec = pl.BlockSpec((2, n, h), lambda i: (0, 0, 0))
    cnt_spec = _full((2, n_pad))

    y2, hr2 = pl.pallas_call(
        functools.partial(_stage_b, n),
        grid=(1,),
        out_shape=[jax.ShapeDtypeStruct((n, h), f32),
                   jax.ShapeDtypeStruct((n, h), f32)],
        in_specs=[acc_spec, cnt_spec, _full((n, h)), _full((1, h)),
                  _full((h, h)), _full((h, h))],
        out_specs=[_full((n, h)), _full((n, h))],
    )(s1, cnt, xr, b1r, W2l, W2r)

    seg2 = _make_seg_sum(n, e, h, False, 2000)
    (s2,) = seg2(y2, src, dst)

    out = pl.pallas_call(
        functools.partial(_stage_c, n),
        grid=(1,),
        out_shape=jax.ShapeDtypeStruct((n, c_out), f32),
        in_specs=[acc_spec, cnt_spec, _full((n, h)), _full((1, h)),
                  _full((c_out, h)), _full((1, c_out))],
        out_specs=_full((n, c_out)),
    )(s2, cnt, hr2, b2r, Wout, boutr)
    return out


# speedup vs baseline: 27.0634x; 1.0917x over previous
import functools

import jax
import jax.numpy as jnp
from jax import lax
from jax.experimental import pallas as pl
from jax.experimental.pallas import tpu as pltpu
from jax.experimental.pallas import tpu_sc as plsc

_NC = 2
_NS = 16
_NW = _NC * _NS


@functools.lru_cache(maxsize=None)
def _make_seg_sum(n, e, h, with_count, chunk):
    epw = e // _NW
    nchunks = epw // chunk
    assert epw * _NW == e and nchunks * chunk == epw and chunk % 16 == 0
    assert n % 16 == 0
    rps = (n + _NS - 1) // _NS
    rps = (rps + 15) & ~15
    n_pad = rps * _NS

    mesh = plsc.VectorSubcoreMesh(core_axis_name="c", subcore_axis_name="s")

    out_type = [jax.ShapeDtypeStruct((_NC, n_pad, h), jnp.float32)]
    scratch = [
        pltpu.VMEM((epw,), jnp.int32),
        pltpu.VMEM((epw,), jnp.int32),
        pltpu.VMEM((chunk, h), jnp.float32),
        pltpu.VMEM((chunk, h), jnp.float32),
        pltpu.VMEM((rps, h), jnp.float32),
        pltpu.VMEM_SHARED((n_pad, h), jnp.float32),
        pltpu.SemaphoreType.DMA,
        pltpu.SemaphoreType.DMA,
        pltpu.SemaphoreType.DMA,
        pltpu.SemaphoreType.DMA,
    ]
    if with_count:
        out_type.append(jax.ShapeDtypeStruct((_NC, n_pad), jnp.float32))
        scratch.append(pltpu.VMEM((chunk,), jnp.float32))
        scratch.append(pltpu.VMEM((rps,), jnp.float32))
        scratch.append(pltpu.VMEM_SHARED((n_pad,), jnp.float32))
        scratch.append(pltpu.SemaphoreType.DMA)
        scratch.append(pltpu.SemaphoreType.DMA)

    def body(y_hbm, src_hbm, dst_hbm, *rest):
        if with_count:
            (out_hbm, cnt_hbm, src_v, dst_v, rv0, rv1, zer_v, acc_sh,
             gs0, gs1, ss0, ss1, ones_v, zc_v, cnt_sh, cs0, cs1) = rest
            csem = (cs0, cs1)
        else:
            (out_hbm, src_v, dst_v, rv0, rv1, zer_v, acc_sh,
             gs0, gs1, ss0, ss1) = rest
        rows_v = (rv0, rv1)
        gsem = (gs0, gs1)
        ssem = (ss0, ss1)
        c = lax.axis_index("c")
        s = lax.axis_index("s")
        wid = s * _NC + c
        base = wid * epw

        icp_s = pltpu.async_copy(src_hbm.at[pl.ds(base, epw)], src_v, ssem[0])
        icp_d = pltpu.async_copy(dst_hbm.at[pl.ds(base, epw)], dst_v, ssem[1])

        zrow = jnp.zeros((16,), jnp.float32)

        def zfill(i, _):
            zer_v[i, :] = zrow
            return 0

        lax.fori_loop(0, rps, zfill, 0)
        pltpu.sync_copy(zer_v, acc_sh.at[pl.ds(s * rps, rps), :])
        orow = jnp.ones((16,), jnp.float32)
        if with_count:
            def czfill(i, _):
                zc_v[pl.ds(i * 16, 16)] = zrow
                return 0

            def ofill(i, _):
                ones_v[pl.ds(i * 16, 16)] = orow
                return 0

            lax.fori_loop(0, rps // 16, czfill, 0)
            lax.fori_loop(0, chunk // 16, ofill, 0)
            pltpu.sync_copy(zc_v, cnt_sh.at[pl.ds(s * rps, rps)])
        plsc.subcore_barrier()
        icp_s.wait()
        icp_d.wait()

        def sl(ref, i):
            return ref.at[pl.ds(i * chunk, chunk)]

        gathers = [pltpu.async_copy(y_hbm.at[sl(src_v, 0)], rows_v[0],
                                    gsem[0]), None]
        scatters = [None, None]
        cscatters = [None, None]
        for i in range(nchunks):
            b = i % 2
            b1 = 1 - b
            gathers[b].wait()
            scatters[b] = pltpu.async_copy(rows_v[b],
                                           acc_sh.at[sl(dst_v, i)],
                                           ssem[b], add=True)
            if with_count:
                cscatters[b] = pltpu.async_copy(ones_v,
                                                cnt_sh.at[sl(dst_v, i)],
                                                csem[b], add=True)
            if i + 1 < nchunks:
                if scatters[b1] is not None:
                    scatters[b1].wait()
                    scatters[b1] = None
                if cscatters[b1] is not None:
                    cscatters[b1].wait()
                    cscatters[b1] = None
                gathers[b1] = pltpu.async_copy(y_hbm.at[sl(src_v, i + 1)],
                                               rows_v[b1], gsem[b1])
        for sc in scatters + cscatters:
            if sc is not None:
                sc.wait()
        plsc.subcore_barrier()

        pltpu.sync_copy(acc_sh.at[pl.ds(s * rps, rps), :],
                        out_hbm.at[c, pl.ds(s * rps, rps), :])
        if with_count:
            pltpu.sync_copy(cnt_sh.at[pl.ds(s * rps, rps)],
                            cnt_hbm.at[c, pl.ds(s * rps, rps)])

    return pl.kernel(body, out_type=out_type, mesh=mesh,
                     scratch_types=scratch,
                     compiler_params=pltpu.CompilerParams(
                         use_tc_tiling_on_sc=False))


def _dot_t(a, w):
    return lax.dot_general(a, w, (((1,), (1,)), ((), ())),
                           preferred_element_type=jnp.float32)


def _stage_a(x_ref, wl_ref, wr_ref, y1_ref, xr_ref):
    x = x_ref[...]
    y1_ref[...] = _dot_t(x, wl_ref[...])
    xr_ref[...] = _dot_t(x, wr_ref[...])


def _stage_b(n, s1_ref, cnt_ref, xr_ref, b1_ref, w2l_ref, w2r_ref,
             y2_ref, hr2_ref):
    cnt = jnp.maximum(cnt_ref[0, :n] + cnt_ref[1, :n], 1.0)[:, None]
    h1 = jax.nn.relu((s1_ref[0] + s1_ref[1]) / cnt + b1_ref[...]
                     + xr_ref[...])
    y2_ref[...] = _dot_t(h1, w2l_ref[...])
    hr2_ref[...] = _dot_t(h1, w2r_ref[...])


def _stage_c(n, s2_ref, cnt_ref, hr2_ref, b2_ref, wout_ref, bout_ref,
             out_ref):
    cnt = jnp.maximum(cnt_ref[0, :n] + cnt_ref[1, :n], 1.0)[:, None]
    h2 = jax.nn.relu((s2_ref[0] + s2_ref[1]) / cnt + b2_ref[...]
                     + hr2_ref[...])
    logits = _dot_t(h2, wout_ref[...]) + bout_ref[...]
    m = jnp.max(logits, axis=-1, keepdims=True)
    lse = jnp.log(jnp.sum(jnp.exp(logits - m), axis=-1, keepdims=True)) + m
    out_ref[...] = logits - lse


def kernel(x, edge_index, W1l, b1, W1r, W2l, b2, W2r, Wout, bout):
    n, d = x.shape
    e = edge_index.shape[1]
    h = W1l.shape[0]
    c_out = Wout.shape[0]

    src = edge_index[0]
    dst = edge_index[1]
    b1r = b1.reshape(1, h)
    b2r = b2.reshape(1, h)
    boutr = bout.reshape(1, c_out)

    f32 = jnp.float32
    y1, xr = pl.pallas_call(
        _stage_a,
        out_shape=[jax.ShapeDtypeStruct((n, h), f32),
                   jax.ShapeDtypeStruct((n, h), f32)],
    )(x, W1l, W1r)

    seg1 = _make_seg_sum(n, e, h, True, 2000)
    s1, cnt = seg1(y1, src, dst)
    n_pad = s1.shape[1]

    def _full(shape):
        nd = len(shape)
        return pl.BlockSpec(shape, lambda i: (0,) * nd)

    acc_spec = pl.BlockSpec((2, n, h), lambda i: (0, 0, 0))
    cnt_spec = _full((2, n_pad))

    y2, hr2 = pl.pallas_call(
        functools.partial(_stage_b, n),
        grid=(1,),
        out_shape=[jax.ShapeDtypeStruct((n, h), f32),
                   jax.ShapeDtypeStruct((n, h), f32)],
        in_specs=[acc_spec, cnt_spec, _full((n, h)), _full((1, h)),
                  _full((h, h)), _full((h, h))],
        out_specs=[_full((n, h)), _full((n, h))],
    )(s1, cnt, xr, b1r, W2l, W2r)

    seg2 = _make_seg_sum(n, e, h, False, 2000)
    (s2,) = seg2(y2, src, dst)

    out = pl.pallas_call(
        functools.partial(_stage_c, n),
        grid=(1,),
        out_shape=jax.ShapeDtypeStruct((n, c_out), f32),
        in_specs=[acc_spec, cnt_spec, _full((n, h)), _full((1, h)),
                  _full((c_out, h)), _full((1, c_out))],
        out_specs=_full((n, c_out)),
    )(s2, cnt, hr2, b2r, Wout, boutr)
    return out
